# baseline (speedup 1.0000x reference)
import dataclasses
import functools

import jax
import jax.numpy as jnp
from jax import lax
from jax.experimental import pallas as pl
from jax.experimental.pallas import tpu as pltpu
from jax.experimental.pallas import tpu_sc as plsc

N_NODES = 10000
DIM = 128
N_EDGES = 320000
BATCH = 4096

NC, NS = 2, 16
NW = NC * NS
E_TILE = N_EDGES // NW
BLK = 125
NGRP = 2
GBLK = 40
CBLK = 125
NPAD = 10240
STRIPE = NPAD // NS
PADROW = NPAD - 1
UI = 2 * BATCH
UI_TILE = UI // NW

_MESH = plsc.VectorSubcoreMesh(core_axis_name="c", subcore_axis_name="s",
                               num_cores=NC, num_subcores=NS)

_CP = pltpu.CompilerParams()
if "needs_layout_passes" in pltpu.CompilerParams.__dataclass_fields__:
    _CP = dataclasses.replace(_CP, needs_layout_passes=False)


@functools.partial(
    pl.kernel,
    out_type=(jax.ShapeDtypeStruct((NC, NPAD), jnp.float32),
              jax.ShapeDtypeStruct((NW, NGRP, GBLK, CBLK), jnp.int32),
              jax.ShapeDtypeStruct((NW, NGRP, GBLK, CBLK), jnp.int32),
              jax.ShapeDtypeStruct((NW, 16), jnp.int32)),
    mesh=_MESH,
    scratch_types=[
        pltpu.VMEM((GBLK, BLK), jnp.int32),
        pltpu.VMEM((128,), jnp.float32),
        pltpu.VMEM((4, 128), jnp.int32),
        pltpu.VMEM((E_TILE,), jnp.int32),
        pltpu.VMEM((E_TILE,), jnp.int32),
        pltpu.VMEM((NPAD,), jnp.float32),
        pltpu.VMEM((NGRP, GBLK, CBLK), jnp.int32),
        pltpu.VMEM((NGRP, GBLK, CBLK), jnp.int32),
        pltpu.VMEM((16,), jnp.int32),
        pltpu.VMEM_SHARED((NPAD,), jnp.float32),
        pltpu.VMEM_SHARED((NPAD,), jnp.float32),
    ],
    compiler_params=_CP,
)
def _deg_kernel(dst4_hbm, srcf_hbm, dstf_hbm, uv_hbm, zeros_hbm,
                deg_hbm, csrc_hbm, cdst_hbm, cnt_hbm,
                dst_v, ones_v, uvb_v, srcf_v, dstf_v, need_v,
                csrc_v, cdst_v, cnt_v, deg_sp, need_sp):
    c = lax.axis_index("c")
    s = lax.axis_index("s")
    w = c * NS + s

    pltpu.sync_copy(zeros_hbm.at[pl.ds(s * STRIPE, STRIPE)],
                    deg_sp.at[pl.ds(s * STRIPE, STRIPE)])
    pltpu.sync_copy(zeros_hbm.at[pl.ds(s * STRIPE, STRIPE)],
                    need_sp.at[pl.ds(s * STRIPE, STRIPE)])

    @pl.loop(0, 128, step=16)
    def _(o):
        ones_v[pl.ds(o, 16)] = jnp.full((16,), 1.0, jnp.float32)

    pltpu.sync_copy(srcf_hbm.at[w], srcf_v)
    pltpu.sync_copy(dstf_hbm.at[w], dstf_v)
    pltpu.sync_copy(uv_hbm.at[2 * s], uvb_v.at[pl.ds(0, 2)])
    pltpu.sync_copy(uv_hbm.at[2 * s + 1], uvb_v.at[pl.ds(2, 2)])
    plsc.subcore_barrier()

    @pl.loop(0, NGRP)
    def _(g):
        pltpu.sync_copy(dst4_hbm.at[w, g], dst_v)

        @pl.loop(0, GBLK)
        def _(b):
            pltpu.sync_copy(ones_v.at[pl.ds(0, BLK)], deg_sp.at[dst_v.at[b]],
                            add=True)

    for r in range(4):
        pltpu.sync_copy(ones_v, need_sp.at[uvb_v.at[r]], add=True)

    plsc.subcore_barrier()
    pltpu.sync_copy(deg_sp.at[pl.ds(s * STRIPE, STRIPE)],
                    deg_hbm.at[c, pl.ds(s * STRIPE, STRIPE)])
    pltpu.sync_copy(need_sp, need_v)

    def step(idx, base16):
        o = idx * 16
        d16 = dstf_v[pl.ds(o, 16)]
        s16 = srcf_v[pl.ds(o, 16)]
        keep = plsc.load_gather(need_v, [d16]) > 0.0
        ki = jnp.where(keep, 1, 0).astype(jnp.int32)
        pos = base16 + plsc.cumsum(ki) - 1
        blkof = pos // CBLK
        cc = pos - blkof * CBLK
        g = jnp.where(blkof >= GBLK, 1, 0).astype(jnp.int32)
        r = blkof - g * GBLK
        plsc.store_scatter(csrc_v, [g, r, cc], s16, mask=keep)
        plsc.store_scatter(cdst_v, [g, r, cc], d16, mask=keep)
        return base16 + plsc.all_reduce_population_count(keep)

    n16 = lax.fori_loop(0, E_TILE // 16, step,
                        jnp.zeros((16,), jnp.int32))
    nblk16 = ((n16 + 2 * CBLK - 1) // (2 * CBLK)) * 2
    end16 = nblk16 * CBLK
    iota = lax.iota(jnp.int32, 16)
    padv = jnp.full((16,), PADROW, jnp.int32)

    @pl.loop(0, 256, step=16)
    def _(o):
        pos = n16 + o + iota
        keep = pos < end16
        blkof = pos // CBLK
        cc = pos - blkof * CBLK
        g = jnp.where(blkof >= GBLK, 1, 0).astype(jnp.int32)
        r = blkof - g * GBLK
        dummy = jnp.full((16,), N_NODES, jnp.int32) + (pos & 127)
        plsc.store_scatter(csrc_v, [g, r, cc], padv, mask=keep)
        plsc.store_scatter(cdst_v, [g, r, cc], dummy, mask=keep)

    cnt_v[pl.ds(0, 16)] = nblk16
    pltpu.sync_copy(csrc_v, csrc_hbm.at[w])
    pltpu.sync_copy(cdst_v, cdst_hbm.at[w])
    pltpu.sync_copy(cnt_v, cnt_hbm.at[w])


def _g2_body(deg_ref, feat_ref, w_ref, g2_ref, rdeg_ref):
    dsum = deg_ref[0] + deg_ref[1]
    rdeg = lax.rsqrt(jnp.maximum(dsum, 1.0))
    rdeg_ref[...] = rdeg
    g = jnp.dot(feat_ref[...], w_ref[...], preferred_element_type=jnp.float32,
                precision=lax.Precision.HIGHEST)
    g2_ref[0:N_NODES, :] = g * rdeg[0:N_NODES]
    g2_ref[N_NODES:NPAD, :] = jnp.zeros((NPAD - N_NODES, DIM), jnp.float32)


_g2_call = pl.pallas_call(
    _g2_body,
    out_shape=(jax.ShapeDtypeStruct((NPAD, DIM), jnp.float32),
               jax.ShapeDtypeStruct((NPAD, 1), jnp.float32)),
)


@functools.partial(
    pl.kernel,
    out_type=jax.ShapeDtypeStruct((NC, NPAD, DIM), jnp.float32),
    mesh=_MESH,
    scratch_types=[
        pltpu.VMEM((GBLK, CBLK), jnp.int32),
        pltpu.VMEM((GBLK, CBLK), jnp.int32),
        pltpu.VMEM((2, CBLK, DIM), jnp.float32),
        pltpu.VMEM((16,), jnp.int32),
        pltpu.VMEM_SHARED((NPAD, DIM), jnp.float32),
        pltpu.SemaphoreType.DMA,
        pltpu.SemaphoreType.DMA,
    ],
    compiler_params=_CP,
)
def _agg_kernel(csrc_hbm, cdst_hbm, cnt_hbm, g2_hbm, zeros_hbm, agg_hbm,
                src_v, dst_v, rows_v, cnt_v, agg_sp, gsem0, gsem1):
    c = lax.axis_index("c")
    s = lax.axis_index("s")
    w = c * NS + s

    pltpu.sync_copy(zeros_hbm.at[pl.ds(s * STRIPE, STRIPE)],
                    agg_sp.at[pl.ds(s * STRIPE, STRIPE)])
    pltpu.sync_copy(cnt_hbm.at[w], cnt_v)
    nb = cnt_v[pl.ds(0, 16)][0]
    plsc.subcore_barrier()

    sems = (gsem0, gsem1)

    def fire(b, buf):
        pltpu.async_copy(g2_hbm.at[src_v.at[b]], rows_v.at[buf], sems[buf])

    def drain(buf):
        pltpu.make_async_copy(g2_hbm.at[src_v.at[0]], rows_v.at[buf],
                              sems[buf]).wait()

    for g in range(NGRP):
        @pl.when(nb > g * GBLK)
        def _():
            pltpu.sync_copy(csrc_hbm.at[w, g], src_v)
            pltpu.sync_copy(cdst_hbm.at[w, g], dst_v)
            ng = jnp.minimum(nb - g * GBLK, GBLK)
            fire(0, 0)
            fire(1, 1)

            for b in range(0, GBLK, 2):
                @pl.when(b < ng)
                def _():
                    drain(0)
                    pltpu.sync_copy(rows_v.at[0], agg_sp.at[dst_v.at[b]],
                                    add=True)

                    @pl.when(b + 2 < ng)
                    def _():
                        fire(b + 2, 0)

                    drain(1)
                    pltpu.sync_copy(rows_v.at[1], agg_sp.at[dst_v.at[b + 1]],
                                    add=True)

                    @pl.when(b + 3 < ng)
                    def _():
                        fire(b + 3, 1)

    plsc.subcore_barrier()
    pltpu.sync_copy(agg_sp.at[pl.ds(s * STRIPE, STRIPE)],
                    agg_hbm.at[c, pl.ds(s * STRIPE, STRIPE)])


def _h_body(agg_ref, rdeg_ref, h_ref):
    ssum = agg_ref[0] + agg_ref[1]
    h_ref[...] = jnp.maximum(ssum * rdeg_ref[...], 0.0)


_h_call = pl.pallas_call(
    _h_body,
    grid=(8,),
    in_specs=[
        pl.BlockSpec((NC, NPAD // 8, DIM), lambda g: (0, g, 0)),
        pl.BlockSpec((NPAD // 8, 1), lambda g: (g, 0)),
    ],
    out_specs=pl.BlockSpec((NPAD // 8, DIM), lambda g: (g, 0)),
    out_shape=jax.ShapeDtypeStruct((NPAD, DIM), jnp.float32),
)


@functools.partial(
    pl.kernel,
    out_type=jax.ShapeDtypeStruct((UI, DIM), jnp.float32),
    mesh=_MESH,
    scratch_types=[
        pltpu.VMEM((2, 128), jnp.int32),
        pltpu.VMEM((UI_TILE, DIM), jnp.float32),
        pltpu.SemaphoreType.DMA,
    ],
)
def _take_kernel(h_hbm, uv_hbm, out_hbm, idx_v, rows_v, sem):
    c = lax.axis_index("c")
    s = lax.axis_index("s")
    w = c * NS + s

    pltpu.sync_copy(uv_hbm.at[w], idx_v)
    cp0 = pltpu.async_copy(h_hbm.at[idx_v.at[0]], rows_v.at[pl.ds(0, 128)], sem)
    cp1 = pltpu.async_copy(h_hbm.at[idx_v.at[1]], rows_v.at[pl.ds(128, 128)], sem)
    cp0.wait()
    cp1.wait()
    pltpu.sync_copy(rows_v, out_hbm.at[pl.ds(w * UI_TILE, UI_TILE)])


def _mlp_body(hui_ref, uw1_ref, ub1_ref, uw2_ref, ub2_ref,
              iw1_ref, ib1_ref, iw2_ref, ib2_ref, user_ref, item_ref):
    hu = hui_ref[0:BATCH, :]
    hi = hui_ref[BATCH:UI, :]
    tu = jnp.maximum(
        jnp.dot(hu, uw1_ref[...], preferred_element_type=jnp.float32,
                precision=lax.Precision.HIGHEST)
        + ub1_ref[...], 0.0)
    user_ref[...] = (jnp.dot(tu, uw2_ref[...], preferred_element_type=jnp.float32,
                             precision=lax.Precision.HIGHEST)
                     + ub2_ref[...])
    ti = jnp.maximum(
        jnp.dot(hi, iw1_ref[...], preferred_element_type=jnp.float32,
                precision=lax.Precision.HIGHEST)
        + ib1_ref[...], 0.0)
    item_ref[...] = (jnp.dot(ti, iw2_ref[...], preferred_element_type=jnp.float32,
                             precision=lax.Precision.HIGHEST)
                     + ib2_ref[...])


_mlp_call = pl.pallas_call(
    _mlp_body,
    out_shape=(jax.ShapeDtypeStruct((BATCH, DIM), jnp.float32),
               jax.ShapeDtypeStruct((BATCH, DIM), jnp.float32)),
)


def kernel(u, i, features, edge_index, W_gcn, uW1, ub1, uW2, ub2,
           iW1, ib1, iW2, ib2):
    dst4 = edge_index[1].reshape(NW, NGRP, GBLK, BLK).astype(jnp.int32)
    srcf = edge_index[0].reshape(NW, E_TILE).astype(jnp.int32)
    dstf = edge_index[1].reshape(NW, E_TILE).astype(jnp.int32)
    uv = jnp.concatenate([u, i]).reshape(NW, 2, 128).astype(jnp.int32)

    zeros_n = jnp.zeros((NPAD,), jnp.float32)
    zeros_nd = jnp.zeros((NPAD, DIM), jnp.float32)

    deg_p, csrc, cdst, cnt = _deg_kernel(dst4, srcf, dstf, uv, zeros_n)
    g2, rdeg = _g2_call(deg_p.reshape(NC, NPAD, 1), features, W_gcn)
    agg_p = _agg_kernel(csrc, cdst, cnt, g2, zeros_nd)
    h = _h_call(agg_p, rdeg)
    hui = _take_kernel(h, uv)
    user, item = _mlp_call(hui, uW1, ub1.reshape(1, DIM), uW2,
                           ub2.reshape(1, DIM), iW1, ib1.reshape(1, DIM),
                           iW2, ib2.reshape(1, DIM))
    return (user, item)


# speedup vs baseline: 1.7599x; 1.7599x over previous
import functools

import jax
import jax.numpy as jnp
from jax import lax
from jax.experimental import pallas as pl
from jax.experimental.pallas import tpu as pltpu
from jax.experimental.pallas import tpu_sc as plsc

N_NODES = 10000
DIM = 128
N_EDGES = 320000
BATCH = 4096

NC, NS = 2, 16
NW = NC * NS
E_TILE = N_EDGES // NW
BLK = 125
NGRP = 2
GBLK = 40
NBLK = NGRP * GBLK
NPAD = 10240
STRIPE = NPAD // NS
UI = 2 * BATCH
UI_TILE = UI // NW

_MESH = plsc.VectorSubcoreMesh(core_axis_name="c", subcore_axis_name="s",
                               num_cores=NC, num_subcores=NS)


@functools.partial(
    pl.kernel,
    out_type=jax.ShapeDtypeStruct((NC, NPAD), jnp.float32),
    mesh=_MESH,
    scratch_types=[
        pltpu.VMEM((GBLK, BLK), jnp.int32),
        pltpu.VMEM((128,), jnp.float32),
        pltpu.VMEM_SHARED((NPAD,), jnp.float32),
    ],
)
def _deg_kernel(dst_hbm, zeros_hbm, deg_hbm, dst_v, ones_v, deg_sp):
    c = lax.axis_index("c")
    s = lax.axis_index("s")
    w = c * NS + s

    pltpu.sync_copy(zeros_hbm.at[pl.ds(s * STRIPE, STRIPE)],
                    deg_sp.at[pl.ds(s * STRIPE, STRIPE)])

    @pl.loop(0, 128, step=16)
    def _(o):
        ones_v[pl.ds(o, 16)] = jnp.full((16,), 1.0, jnp.float32)

    plsc.subcore_barrier()

    @pl.loop(0, NGRP)
    def _(g):
        pltpu.sync_copy(dst_hbm.at[w, g], dst_v)

        @pl.loop(0, GBLK)
        def _(b):
            pltpu.sync_copy(ones_v.at[pl.ds(0, BLK)], deg_sp.at[dst_v.at[b]],
                            add=True)

    plsc.subcore_barrier()
    pltpu.sync_copy(deg_sp.at[pl.ds(s * STRIPE, STRIPE)],
                    deg_hbm.at[c, pl.ds(s * STRIPE, STRIPE)])


def _g2_body(deg_ref, feat_ref, w_ref, g2_ref, rdeg_ref):
    dsum = deg_ref[0] + deg_ref[1]
    rdeg = lax.rsqrt(jnp.maximum(dsum, 1.0))
    rdeg_ref[...] = rdeg
    g = jnp.dot(feat_ref[...], w_ref[...], preferred_element_type=jnp.float32,
                precision=lax.Precision.HIGHEST)
    g2_ref[0:N_NODES, :] = g * rdeg[0:N_NODES]
    g2_ref[N_NODES:NPAD, :] = jnp.zeros((NPAD - N_NODES, DIM), jnp.float32)


_g2_call = pl.pallas_call(
    _g2_body,
    out_shape=(jax.ShapeDtypeStruct((NPAD, DIM), jnp.float32),
               jax.ShapeDtypeStruct((NPAD, 1), jnp.float32)),
)


@functools.partial(
    pl.kernel,
    out_type=jax.ShapeDtypeStruct((NC, NPAD, DIM), jnp.float32),
    mesh=_MESH,
    scratch_types=[
        pltpu.VMEM((GBLK, BLK), jnp.int32),
        pltpu.VMEM((GBLK, BLK), jnp.int32),
        pltpu.VMEM((2, BLK, DIM), jnp.float32),
        pltpu.VMEM_SHARED((NPAD, DIM), jnp.float32),
        pltpu.SemaphoreType.DMA,
        pltpu.SemaphoreType.DMA,
    ],
)
def _agg_kernel(src_hbm, dst_hbm, g2_hbm, zeros_hbm, agg_hbm,
                src_v, dst_v, rows_v, agg_sp, gsem0, gsem1):
    c = lax.axis_index("c")
    s = lax.axis_index("s")
    w = c * NS + s

    pltpu.sync_copy(zeros_hbm.at[pl.ds(s * STRIPE, STRIPE)],
                    agg_sp.at[pl.ds(s * STRIPE, STRIPE)])
    plsc.subcore_barrier()

    sems = (gsem0, gsem1)

    def fire(b, buf):
        pltpu.async_copy(g2_hbm.at[src_v.at[b]], rows_v.at[buf], sems[buf])

    def drain(buf):
        pltpu.make_async_copy(g2_hbm.at[src_v.at[0]], rows_v.at[buf],
                              sems[buf]).wait()

    @pl.loop(0, NGRP)
    def _(g):
        pltpu.sync_copy(src_hbm.at[w, g], src_v)
        pltpu.sync_copy(dst_hbm.at[w, g], dst_v)
        fire(0, 0)
        fire(1, 1)

        @pl.loop(0, GBLK, step=2)
        def _(b):
            drain(0)
            pltpu.sync_copy(rows_v.at[0], agg_sp.at[dst_v.at[b]], add=True)

            @pl.when(b + 2 < GBLK)
            def _():
                fire(b + 2, 0)

            drain(1)
            pltpu.sync_copy(rows_v.at[1], agg_sp.at[dst_v.at[b + 1]], add=True)

            @pl.when(b + 3 < GBLK)
            def _():
                fire(b + 3, 1)

    plsc.subcore_barrier()
    pltpu.sync_copy(agg_sp.at[pl.ds(s * STRIPE, STRIPE)],
                    agg_hbm.at[c, pl.ds(s * STRIPE, STRIPE)])


def _h_body(agg_ref, rdeg_ref, h_ref):
    ssum = agg_ref[0] + agg_ref[1]
    h_ref[...] = jnp.maximum(ssum * rdeg_ref[...], 0.0)


_h_call = pl.pallas_call(
    _h_body,
    grid=(8,),
    in_specs=[
        pl.BlockSpec((NC, NPAD // 8, DIM), lambda g: (0, g, 0)),
        pl.BlockSpec((NPAD // 8, 1), lambda g: (g, 0)),
    ],
    out_specs=pl.BlockSpec((NPAD // 8, DIM), lambda g: (g, 0)),
    out_shape=jax.ShapeDtypeStruct((NPAD, DIM), jnp.float32),
)


@functools.partial(
    pl.kernel,
    out_type=jax.ShapeDtypeStruct((UI, DIM), jnp.float32),
    mesh=_MESH,
    scratch_types=[
        pltpu.VMEM((2, 128), jnp.int32),
        pltpu.VMEM((UI_TILE, DIM), jnp.float32),
        pltpu.SemaphoreType.DMA,
    ],
)
def _take_kernel(h_hbm, uv_hbm, out_hbm, idx_v, rows_v, sem):
    c = lax.axis_index("c")
    s = lax.axis_index("s")
    w = c * NS + s

    pltpu.sync_copy(uv_hbm.at[w], idx_v)
    cp0 = pltpu.async_copy(h_hbm.at[idx_v.at[0]], rows_v.at[pl.ds(0, 128)], sem)
    cp1 = pltpu.async_copy(h_hbm.at[idx_v.at[1]], rows_v.at[pl.ds(128, 128)], sem)
    cp0.wait()
    cp1.wait()
    pltpu.sync_copy(rows_v, out_hbm.at[pl.ds(w * UI_TILE, UI_TILE)])


def _mlp_body(hui_ref, uw1_ref, ub1_ref, uw2_ref, ub2_ref,
              iw1_ref, ib1_ref, iw2_ref, ib2_ref, user_ref, item_ref):
    hu = hui_ref[0:BATCH, :]
    hi = hui_ref[BATCH:UI, :]
    tu = jnp.maximum(
        jnp.dot(hu, uw1_ref[...], preferred_element_type=jnp.float32,
                precision=lax.Precision.HIGHEST)
        + ub1_ref[...], 0.0)
    user_ref[...] = (jnp.dot(tu, uw2_ref[...], preferred_element_type=jnp.float32,
                             precision=lax.Precision.HIGHEST)
                     + ub2_ref[...])
    ti = jnp.maximum(
        jnp.dot(hi, iw1_ref[...], preferred_element_type=jnp.float32,
                precision=lax.Precision.HIGHEST)
        + ib1_ref[...], 0.0)
    item_ref[...] = (jnp.dot(ti, iw2_ref[...], preferred_element_type=jnp.float32,
                             precision=lax.Precision.HIGHEST)
                     + ib2_ref[...])


_mlp_call = pl.pallas_call(
    _mlp_body,
    out_shape=(jax.ShapeDtypeStruct((BATCH, DIM), jnp.float32),
               jax.ShapeDtypeStruct((BATCH, DIM), jnp.float32)),
)


def kernel(u, i, features, edge_index, W_gcn, uW1, ub1, uW2, ub2,
           iW1, ib1, iW2, ib2):
    src = edge_index[0].reshape(NW, NGRP, GBLK, BLK).astype(jnp.int32)
    dst = edge_index[1].reshape(NW, NGRP, GBLK, BLK).astype(jnp.int32)
    uv = jnp.concatenate([u, i]).reshape(NW, 2, 128).astype(jnp.int32)

    zeros_n = jnp.zeros((NPAD,), jnp.float32)
    zeros_nd = jnp.zeros((NPAD, DIM), jnp.float32)

    deg_p = _deg_kernel(dst, zeros_n)
    g2, rdeg = _g2_call(deg_p.reshape(NC, NPAD, 1), features, W_gcn)
    agg_p = _agg_kernel(src, dst, g2, zeros_nd)
    h = _h_call(agg_p, rdeg)
    hui = _take_kernel(h, uv)
    user, item = _mlp_call(hui, uW1, ub1.reshape(1, DIM), uW2,
                           ub2.reshape(1, DIM), iW1, ib1.reshape(1, DIM),
                           iW2, ib2.reshape(1, DIM))
    return (user, item)
